# Initial kernel scaffold; baseline (speedup 1.0000x reference)
import jax
import jax.numpy as jnp
from jax.experimental import pallas as pl


def kernel(b, C, nb, nC, W, context_weights):
    raise NotImplementedError("write your pallas kernel here")


# speedup vs baseline: 1.9700x; 1.9700x over previous
import functools

import jax
import jax.numpy as jnp
from jax import lax
from jax.experimental import pallas as pl
from jax.experimental.pallas import tpu as pltpu
from jax.experimental.pallas import tpu_sc as plsc

VOCAB = 1000000
EMBED = 64
CTX = 20
NLAB = 5
BATCH = 4096

NC = 2
NS = 16
NW = NC * NS
ROWS_PER_W = BATCH // NW
R = 8
NCHUNK = ROWS_PER_W // R
LANES = 16
NK = EMBED // LANES


def _rsqrt_newton(d):
    i = lax.bitcast_convert_type(d, jnp.int32)
    i = jnp.int32(0x5F3759DF) - lax.shift_right_arithmetic(i, 1)
    y = lax.bitcast_convert_type(i, jnp.float32)
    for _ in range(3):
        y = y * (jnp.float32(1.5) - jnp.float32(0.5) * d * y * y)
    return y


def _sc_kernel(w_hbm, bf_hbm, c2_hbm, nbf_hbm, nc2_hbm, wb_hbm, out_hbm,
               idx_b, idx_nb, idx_c, idx_nc,
               buf_b, buf_nb, buf_c, buf_nc,
               wb_v, out_v, sem):
    wid = lax.axis_index("s") * NC + lax.axis_index("c")

    pltpu.sync_copy(wb_hbm, wb_v)

    iota = lax.iota(jnp.int32, LANES)
    lane0 = iota == 0
    lane1 = iota == 1
    lane2 = iota == 2
    zeros = jnp.zeros((LANES,), jnp.float32)
    ones = jnp.ones((LANES,), jnp.float32)

    def chunk_body(g, lvec):
        base = wid * ROWS_PER_W + g * R

        pltpu.sync_copy(bf_hbm.at[pl.ds(base * NLAB, R * NLAB)], idx_b)
        pltpu.sync_copy(nbf_hbm.at[pl.ds(base * NLAB, R * NLAB)], idx_nb)
        pltpu.sync_copy(c2_hbm.at[pl.ds(base, R)], idx_c)
        pltpu.sync_copy(nc2_hbm.at[pl.ds(base, R)], idx_nc)

        descs = [
            pltpu.async_copy(w_hbm.at[idx_b], buf_b, sem),
            pltpu.async_copy(w_hbm.at[idx_nb], buf_nb, sem),
        ]
        for r in range(R):
            descs.append(pltpu.async_copy(
                w_hbm.at[idx_c.at[r]], buf_c.at[pl.ds(r * CTX * NLAB, CTX * NLAB)], sem))
            descs.append(pltpu.async_copy(
                w_hbm.at[idx_nc.at[r]], buf_nc.at[pl.ds(r * CTX * NLAB, CTX * NLAB)], sem))
        for d in descs:
            d.wait()

        def row_body(r, lvec):
            be = [zeros] * NK
            nbe = [zeros] * NK
            for l in range(NLAB):
                for k in range(NK):
                    be[k] = be[k] + buf_b[r * NLAB + l, pl.ds(k * LANES, LANES)]
                    nbe[k] = nbe[k] + buf_nb[r * NLAB + l, pl.ds(k * LANES, LANES)]

            def ctx_body(c, carry):
                acc = list(carry)
                wv = wb_v[c, :]
                row0 = r * CTX * NLAB + c * NLAB
                for k in range(NK):
                    p = buf_c[row0, pl.ds(k * LANES, LANES)]
                    for l in range(1, NLAB):
                        p = p + buf_c[row0 + l, pl.ds(k * LANES, LANES)]
                    acc[k] = acc[k] + wv * p
                for k in range(NK):
                    p = buf_nc[row0, pl.ds(k * LANES, LANES)]
                    for l in range(1, NLAB):
                        p = p + buf_nc[row0 + l, pl.ds(k * LANES, LANES)]
                    acc[NK + k] = acc[NK + k] + wv * p
                return tuple(acc)

            hs = lax.fori_loop(0, CTX, ctx_body, (zeros,) * (2 * NK))
            h = hs[:NK]
            nh = hs[NK:]

            def dot(a, b):
                v = a[0] * b[0]
                for k in range(1, NK):
                    v = v + a[k] * b[k]
                return jnp.sum(v)

            d_bh = dot(be, h)
            d_nbh = dot(nbe, h)
            d_bnh = dot(be, nh)
            q_b = dot(be, be)
            q_h = dot(h, h)
            q_nb = dot(nbe, nbe)
            q_nh = dot(nh, nh)

            num = jnp.where(lane0, -d_bh,
                            jnp.where(lane1, jnp.float32(0.5) * d_nbh,
                                      jnp.where(lane2, jnp.float32(0.5) * d_bnh,
                                                zeros)))
            den = jnp.where(lane0, q_b * q_h,
                            jnp.where(lane1, q_nb * q_h,
                                      jnp.where(lane2, q_b * q_nh, ones)))
            den = jnp.maximum(den, jnp.float32(1e-30))
            return lvec + num * _rsqrt_newton(den)

        return lax.fori_loop(0, R, row_body, lvec)

    lvec = lax.fori_loop(0, NCHUNK, chunk_body, zeros)
    out_v[...] = lvec * jnp.float32(1.0 / BATCH)
    pltpu.sync_copy(out_v, out_hbm.at[wid])


@jax.jit
def kernel(b, C, nb, nC, W, context_weights):
    bf = b.reshape(BATCH * NLAB)
    nbf = nb.reshape(BATCH * NLAB)
    c2 = C.reshape(BATCH, CTX * NLAB)
    nc2 = nC.reshape(BATCH, CTX * NLAB)
    wb = jnp.broadcast_to(context_weights[:, None], (CTX, LANES))

    mesh = plsc.VectorSubcoreMesh(core_axis_name="c", subcore_axis_name="s",
                                  num_cores=NC, num_subcores=NS)
    run = pl.kernel(
        _sc_kernel,
        out_type=jax.ShapeDtypeStruct((NW, LANES), jnp.float32),
        mesh=mesh,
        compiler_params=pltpu.CompilerParams(needs_layout_passes=False,
                                             use_tc_tiling_on_sc=False),
        scratch_types=[
            pltpu.VMEM((R * NLAB,), jnp.int32),
            pltpu.VMEM((R * NLAB,), jnp.int32),
            pltpu.VMEM((R, CTX * NLAB), jnp.int32),
            pltpu.VMEM((R, CTX * NLAB), jnp.int32),
            pltpu.VMEM((R * NLAB, EMBED), jnp.float32),
            pltpu.VMEM((R * NLAB, EMBED), jnp.float32),
            pltpu.VMEM((R * CTX * NLAB, EMBED), jnp.float32),
            pltpu.VMEM((R * CTX * NLAB, EMBED), jnp.float32),
            pltpu.VMEM((CTX, LANES), jnp.float32),
            pltpu.VMEM((LANES,), jnp.float32),
            pltpu.SemaphoreType.DMA,
        ],
    )
    partials = run(W, bf, c2, nbf, nc2, wb)
    return jnp.sum(partials)


# speedup vs baseline: 2.2055x; 1.1195x over previous
import jax
import jax.numpy as jnp
from jax import lax
from jax.experimental import pallas as pl
from jax.experimental.pallas import tpu as pltpu
from jax.experimental.pallas import tpu_sc as plsc

VOCAB = 1000000
EMBED = 64
CTX = 20
NLAB = 5
BATCH = 4096

NC = 2
NS = 16
NW = NC * NS
ROWS_PER_W = BATCH // NW
R = 4
NCHUNK = ROWS_PER_W // R
CL = CTX * NLAB
LANES = 16
NK = EMBED // LANES


def _rsqrt_newton(d):
    i = lax.bitcast_convert_type(d, jnp.int32)
    i = jnp.int32(0x5F3759DF) - lax.shift_right_arithmetic(i, 1)
    y = lax.bitcast_convert_type(i, jnp.float32)
    for _ in range(3):
        y = y * (jnp.float32(1.5) - jnp.float32(0.5) * d * y * y)
    return y


def _sc_kernel(w_hbm, b2_hbm, c2_hbm, nb2_hbm, nc2_hbm, wb_hbm, out_hbm,
               idx_b0, idx_nb0, idx_c0, idx_nc0,
               idx_b1, idx_nb1, idx_c1, idx_nc1,
               buf_b0, buf_nb0, buf_c0, buf_nc0,
               buf_b1, buf_nb1, buf_c1, buf_nc1,
               wb_v, out_v, semi0, semi1, semd0, semd1):
    wid = lax.axis_index("s") * NC + lax.axis_index("c")
    wbase = wid * ROWS_PER_W

    pltpu.sync_copy(wb_hbm, wb_v)

    sets = (
        (idx_b0, idx_nb0, idx_c0, idx_nc0, buf_b0, buf_nb0, buf_c0, buf_nc0,
         semi0, semd0),
        (idx_b1, idx_nb1, idx_c1, idx_nc1, buf_b1, buf_nb1, buf_c1, buf_nc1,
         semi1, semd1),
    )

    def idx_copies(p, ch):
        ib, inb, ic, inc, _, _, _, _, semi, _ = sets[p]
        base = wbase + ch * R
        return [
            (b2_hbm.at[pl.ds(base, R)], ib, semi),
            (nb2_hbm.at[pl.ds(base, R)], inb, semi),
            (c2_hbm.at[pl.ds(base, R)], ic, semi),
            (nc2_hbm.at[pl.ds(base, R)], inc, semi),
        ]

    def data_copies(p):
        ib, inb, ic, inc, bb, bnb, bc, bnc, _, semd = sets[p]
        out = []
        for r in range(R):
            out.append((w_hbm.at[ic.at[r]], bc.at[pl.ds(r * CL, CL)], semd))
            out.append((w_hbm.at[inc.at[r]], bnc.at[pl.ds(r * CL, CL)], semd))
            out.append((w_hbm.at[ib.at[r]], bb.at[pl.ds(r * NLAB, NLAB)], semd))
            out.append((w_hbm.at[inb.at[r]], bnb.at[pl.ds(r * NLAB, NLAB)], semd))
        return out

    def stage(p, ch):
        for s, d, sem in idx_copies(p, ch):
            pltpu.async_copy(s, d, sem)

    def fire(p, ch):
        for s, d, sem in idx_copies(p, ch):
            pltpu.make_async_copy(s, d, sem).wait()
        for s, d, sem in data_copies(p):
            pltpu.async_copy(s, d, sem)

    iota = lax.iota(jnp.int32, LANES)
    lane0 = iota == 0
    lane1 = iota == 1
    lane2 = iota == 2
    zeros = jnp.zeros((LANES,), jnp.float32)
    ones = jnp.ones((LANES,), jnp.float32)

    def compute(p, lvec):
        _, _, _, _, bb, bnb, bc, bnc, _, _ = sets[p]
        for s, d, sem in data_copies(p):
            pltpu.make_async_copy(s, d, sem).wait()

        def row_body(r, lvec):
            be = [zeros] * NK
            nbe = [zeros] * NK
            for l in range(NLAB):
                for k in range(NK):
                    be[k] = be[k] + bb[r * NLAB + l, pl.ds(k * LANES, LANES)]
                    nbe[k] = nbe[k] + bnb[r * NLAB + l, pl.ds(k * LANES, LANES)]

            def ctx_body(c, carry):
                acc = list(carry)
                wv = wb_v[c, :]
                row0 = r * CL + c * NLAB
                for k in range(NK):
                    q = bc[row0, pl.ds(k * LANES, LANES)]
                    for l in range(1, NLAB):
                        q = q + bc[row0 + l, pl.ds(k * LANES, LANES)]
                    acc[k] = acc[k] + wv * q
                for k in range(NK):
                    q = bnc[row0, pl.ds(k * LANES, LANES)]
                    for l in range(1, NLAB):
                        q = q + bnc[row0 + l, pl.ds(k * LANES, LANES)]
                    acc[NK + k] = acc[NK + k] + wv * q
                return tuple(acc)

            hs = lax.fori_loop(0, CTX, ctx_body, (zeros,) * (2 * NK))
            h = hs[:NK]
            nh = hs[NK:]

            def dot(a, b):
                v = a[0] * b[0]
                for k in range(1, NK):
                    v = v + a[k] * b[k]
                return jnp.sum(v)

            d_bh = dot(be, h)
            d_nbh = dot(nbe, h)
            d_bnh = dot(be, nh)
            q_b = dot(be, be)
            q_h = dot(h, h)
            q_nb = dot(nbe, nbe)
            q_nh = dot(nh, nh)

            num = jnp.where(lane0, -d_bh,
                            jnp.where(lane1, jnp.float32(0.5) * d_nbh,
                                      jnp.where(lane2, jnp.float32(0.5) * d_bnh,
                                                zeros)))
            den = jnp.where(lane0, q_b * q_h,
                            jnp.where(lane1, q_nb * q_h,
                                      jnp.where(lane2, q_b * q_nh, ones)))
            den = jnp.maximum(den, jnp.float32(1e-30))
            return lvec + num * _rsqrt_newton(den)

        return lax.fori_loop(0, R, row_body, lvec)

    stage(0, 0)
    fire(0, 0)
    stage(1, 1)

    def pipe_body(g, lvec):
        fire(1, 2 * g + 1)
        lvec = compute(0, lvec)

        @pl.when(g < NCHUNK // 2 - 1)
        def _():
            stage(0, 2 * g + 2)
            fire(0, 2 * g + 2)
            stage(1, 2 * g + 3)

        return compute(1, lvec)

    lvec = lax.fori_loop(0, NCHUNK // 2, pipe_body, zeros)
    out_v[...] = lvec * jnp.float32(1.0 / BATCH)
    pltpu.sync_copy(out_v, out_hbm.at[wid])


@jax.jit
def kernel(b, C, nb, nC, W, context_weights):
    c2 = C.reshape(BATCH, CL)
    nc2 = nC.reshape(BATCH, CL)
    wb = jnp.broadcast_to(context_weights[:, None], (CTX, LANES))

    mesh = plsc.VectorSubcoreMesh(core_axis_name="c", subcore_axis_name="s",
                                  num_cores=NC, num_subcores=NS)
    run = pl.kernel(
        _sc_kernel,
        out_type=jax.ShapeDtypeStruct((NW, LANES), jnp.float32),
        mesh=mesh,
        compiler_params=pltpu.CompilerParams(needs_layout_passes=False,
                                             use_tc_tiling_on_sc=False),
        scratch_types=[
            pltpu.VMEM((R, NLAB), jnp.int32),
            pltpu.VMEM((R, NLAB), jnp.int32),
            pltpu.VMEM((R, CL), jnp.int32),
            pltpu.VMEM((R, CL), jnp.int32),
            pltpu.VMEM((R, NLAB), jnp.int32),
            pltpu.VMEM((R, NLAB), jnp.int32),
            pltpu.VMEM((R, CL), jnp.int32),
            pltpu.VMEM((R, CL), jnp.int32),
            pltpu.VMEM((R * NLAB, EMBED), jnp.float32),
            pltpu.VMEM((R * NLAB, EMBED), jnp.float32),
            pltpu.VMEM((R * CL, EMBED), jnp.float32),
            pltpu.VMEM((R * CL, EMBED), jnp.float32),
            pltpu.VMEM((R * NLAB, EMBED), jnp.float32),
            pltpu.VMEM((R * NLAB, EMBED), jnp.float32),
            pltpu.VMEM((R * CL, EMBED), jnp.float32),
            pltpu.VMEM((R * CL, EMBED), jnp.float32),
            pltpu.VMEM((CTX, LANES), jnp.float32),
            pltpu.VMEM((LANES,), jnp.float32),
            pltpu.SemaphoreType.DMA,
            pltpu.SemaphoreType.DMA,
            pltpu.SemaphoreType.DMA,
            pltpu.SemaphoreType.DMA,
        ],
    )
    partials = run(W, b, c2, nb, nc2, wb)
    return jnp.sum(partials)


# speedup vs baseline: 2.2095x; 1.0018x over previous
import jax
import jax.numpy as jnp
from jax import lax
from jax.experimental import layout as jlayout
from jax.experimental import pallas as pl
from jax.experimental.pallas import tpu as pltpu
from jax.experimental.pallas import tpu_sc as plsc

VOCAB = 1000000
EMBED = 64
CTX = 20
NLAB = 5
BATCH = 4096

NC = 2
NS = 16
NW = NC * NS
ROWS_PER_W = BATCH // NW
R = 4
NCHUNK = ROWS_PER_W // R
CL = CTX * NLAB
LANES = 16
NK = EMBED // LANES


def _rsqrt_newton(d):
    i = lax.bitcast_convert_type(d, jnp.int32)
    i = jnp.int32(0x5F3759DF) - lax.shift_right_arithmetic(i, 1)
    y = lax.bitcast_convert_type(i, jnp.float32)
    for _ in range(3):
        y = y * (jnp.float32(1.5) - jnp.float32(0.5) * d * y * y)
    return y


def _sc_kernel(w_hbm, b2_hbm, c2_hbm, nb2_hbm, nc2_hbm, wb_hbm, out_hbm,
               idx_b0, idx_nb0, idx_c0, idx_nc0,
               idx_b1, idx_nb1, idx_c1, idx_nc1,
               buf_b0, buf_nb0, buf_c0, buf_nc0,
               buf_b1, buf_nb1, buf_c1, buf_nc1,
               wb_v, out_v, semi0, semi1, semd0, semd1):
    wid = lax.axis_index("s") * NC + lax.axis_index("c")
    wbase = wid * ROWS_PER_W

    pltpu.sync_copy(wb_hbm, wb_v)

    sets = (
        (idx_b0, idx_nb0, idx_c0, idx_nc0, buf_b0, buf_nb0, buf_c0, buf_nc0,
         semi0, semd0),
        (idx_b1, idx_nb1, idx_c1, idx_nc1, buf_b1, buf_nb1, buf_c1, buf_nc1,
         semi1, semd1),
    )

    def idx_copies(p, ch):
        ib, inb, ic, inc, _, _, _, _, semi, _ = sets[p]
        base = wbase + ch * R
        return [
            (b2_hbm.at[pl.ds(base, R)], ib, semi),
            (nb2_hbm.at[pl.ds(base, R)], inb, semi),
            (c2_hbm.at[pl.ds(base, R)], ic, semi),
            (nc2_hbm.at[pl.ds(base, R)], inc, semi),
        ]

    def data_copies(p):
        ib, inb, ic, inc, bb, bnb, bc, bnc, _, semd = sets[p]
        out = []
        for r in range(R):
            out.append((w_hbm.at[ic.at[r]], bc.at[pl.ds(r * CL, CL)], semd))
            out.append((w_hbm.at[inc.at[r]], bnc.at[pl.ds(r * CL, CL)], semd))
            out.append((w_hbm.at[ib.at[r]], bb.at[pl.ds(r * NLAB, NLAB)], semd))
            out.append((w_hbm.at[inb.at[r]], bnb.at[pl.ds(r * NLAB, NLAB)], semd))
        return out

    def stage(p, ch):
        for s, d, sem in idx_copies(p, ch):
            pltpu.async_copy(s, d, sem)

    def fire(p, ch):
        for s, d, sem in idx_copies(p, ch):
            pltpu.make_async_copy(s, d, sem).wait()
        for s, d, sem in data_copies(p):
            pltpu.async_copy(s, d, sem)

    iota = lax.iota(jnp.int32, LANES)
    lane0 = iota == 0
    lane1 = iota == 1
    lane2 = iota == 2
    zeros = jnp.zeros((LANES,), jnp.float32)
    ones = jnp.ones((LANES,), jnp.float32)

    def compute(p, lvec):
        _, _, _, _, bb, bnb, bc, bnc, _, _ = sets[p]
        for s, d, sem in data_copies(p):
            pltpu.make_async_copy(s, d, sem).wait()

        def row_body(r, lvec):
            be = [zeros] * NK
            nbe = [zeros] * NK
            for l in range(NLAB):
                for k in range(NK):
                    be[k] = be[k] + bb[r * NLAB + l, pl.ds(k * LANES, LANES)]
                    nbe[k] = nbe[k] + bnb[r * NLAB + l, pl.ds(k * LANES, LANES)]

            def ctx_body(c, carry):
                acc = list(carry)
                wv = wb_v[c, :]
                row0 = r * CL + c * NLAB
                for k in range(NK):
                    q = bc[row0, pl.ds(k * LANES, LANES)]
                    for l in range(1, NLAB):
                        q = q + bc[row0 + l, pl.ds(k * LANES, LANES)]
                    acc[k] = acc[k] + wv * q
                for k in range(NK):
                    q = bnc[row0, pl.ds(k * LANES, LANES)]
                    for l in range(1, NLAB):
                        q = q + bnc[row0 + l, pl.ds(k * LANES, LANES)]
                    acc[NK + k] = acc[NK + k] + wv * q
                return tuple(acc)

            hs = lax.fori_loop(0, CTX, ctx_body, (zeros,) * (2 * NK))
            h = hs[:NK]
            nh = hs[NK:]

            def dot(a, b):
                v = a[0] * b[0]
                for k in range(1, NK):
                    v = v + a[k] * b[k]
                return jnp.sum(v)

            d_bh = dot(be, h)
            d_nbh = dot(nbe, h)
            d_bnh = dot(be, nh)
            q_b = dot(be, be)
            q_h = dot(h, h)
            q_nb = dot(nbe, nbe)
            q_nh = dot(nh, nh)

            num = jnp.where(lane0, -d_bh,
                            jnp.where(lane1, jnp.float32(0.5) * d_nbh,
                                      jnp.where(lane2, jnp.float32(0.5) * d_bnh,
                                                zeros)))
            den = jnp.where(lane0, q_b * q_h,
                            jnp.where(lane1, q_nb * q_h,
                                      jnp.where(lane2, q_b * q_nh, ones)))
            den = jnp.maximum(den, jnp.float32(1e-30))
            return lvec + num * _rsqrt_newton(den)

        return lax.fori_loop(0, R, row_body, lvec)

    stage(0, 0)
    fire(0, 0)
    stage(1, 1)

    def pipe_body(g, lvec):
        fire(1, 2 * g + 1)
        lvec = compute(0, lvec)

        @pl.when(g < NCHUNK // 2 - 1)
        def _():
            stage(0, 2 * g + 2)
            fire(0, 2 * g + 2)
            stage(1, 2 * g + 3)

        return compute(1, lvec)

    lvec = lax.fori_loop(0, NCHUNK // 2, pipe_body, zeros)
    out_v[...] = lvec * jnp.float32(1.0 / BATCH)
    pltpu.sync_copy(out_v, out_hbm.at[wid])


@jax.jit
def kernel(b, C, nb, nC, W, context_weights):
    W = jlayout.with_layout_constraint(W, jlayout.Layout((1, 0), tiling=()))
    c2 = C.reshape(BATCH, CL)
    nc2 = nC.reshape(BATCH, CL)
    wb = jnp.broadcast_to(context_weights[:, None], (CTX, LANES))

    mesh = plsc.VectorSubcoreMesh(core_axis_name="c", subcore_axis_name="s",
                                  num_cores=NC, num_subcores=NS)
    run = pl.kernel(
        _sc_kernel,
        out_type=jax.ShapeDtypeStruct((NW, LANES), jnp.float32),
        mesh=mesh,
        compiler_params=pltpu.CompilerParams(needs_layout_passes=False,
                                             use_tc_tiling_on_sc=False),
        scratch_types=[
            pltpu.VMEM((R, NLAB), jnp.int32),
            pltpu.VMEM((R, NLAB), jnp.int32),
            pltpu.VMEM((R, CL), jnp.int32),
            pltpu.VMEM((R, CL), jnp.int32),
            pltpu.VMEM((R, NLAB), jnp.int32),
            pltpu.VMEM((R, NLAB), jnp.int32),
            pltpu.VMEM((R, CL), jnp.int32),
            pltpu.VMEM((R, CL), jnp.int32),
            pltpu.VMEM((R * NLAB, EMBED), jnp.float32),
            pltpu.VMEM((R * NLAB, EMBED), jnp.float32),
            pltpu.VMEM((R * CL, EMBED), jnp.float32),
            pltpu.VMEM((R * CL, EMBED), jnp.float32),
            pltpu.VMEM((R * NLAB, EMBED), jnp.float32),
            pltpu.VMEM((R * NLAB, EMBED), jnp.float32),
            pltpu.VMEM((R * CL, EMBED), jnp.float32),
            pltpu.VMEM((R * CL, EMBED), jnp.float32),
            pltpu.VMEM((CTX, LANES), jnp.float32),
            pltpu.VMEM((LANES,), jnp.float32),
            pltpu.SemaphoreType.DMA,
            pltpu.SemaphoreType.DMA,
            pltpu.SemaphoreType.DMA,
            pltpu.SemaphoreType.DMA,
        ],
    )
    partials = run(W, b, c2, nb, nc2, wb)
    return jnp.sum(partials)
